# Initial kernel scaffold; baseline (speedup 1.0000x reference)
import jax
import jax.numpy as jnp
from jax.experimental import pallas as pl


def kernel(edge_index, edge_attr, path_emb, sim_w, W_l, b_l, W_r, b_r, W_e, att, bias, lin_W, lin_b):
    raise NotImplementedError("write your pallas kernel here")


# speedup vs baseline: 15.8716x; 15.8716x over previous
import jax
import jax.numpy as jnp
from jax import lax
from jax.experimental import pallas as pl
from jax.experimental.pallas import tpu as pltpu
from jax.experimental.pallas import tpu_sc as plsc

N = 10000
E = 320000
EMB = 128
HID = 128
H = 3

NC = 2
NS = 16
NW = NC * NS
EPW = E // NW
C = 80
NCHUNK = EPW // C

ROWL = 512
ROWR = 384

NB = 400
ACCW = 16
NPAD = 10240
NR = NPAD // NS


def _prep_body(x_ref, wl_ref, bl_ref, wr_ref, br_ref, linw_ref,
               xl_ref, xr_ref):
  x = x_ref[...]
  xl = jnp.dot(x, wl_ref[...], preferred_element_type=jnp.float32) + bl_ref[...]
  xr = jnp.dot(x, wr_ref[...], preferred_element_type=jnp.float32) + br_ref[...]
  sl = [jnp.dot(xl[:, h * HID:(h + 1) * HID], linw_ref[...],
                preferred_element_type=jnp.float32) for h in range(H)]
  tail = jnp.concatenate(sl + [jnp.zeros((NB, ROWL - H * HID - H), jnp.float32)],
                         axis=1)
  xl_ref[...] = jnp.concatenate([xl, tail], axis=1)
  xr_ref[...] = xr


def _fused_body(attr_ref, simw_ref, out_ref):
  sw = simw_ref[...]
  sw = sw - jnp.max(sw)
  e = jnp.exp(sw)
  w = e / jnp.sum(e)
  f = jnp.sum(attr_ref[...] * w, axis=1, keepdims=True)
  out_ref[...] = jnp.broadcast_to(f, (f.shape[0], ACCW))


def _sc_body(xl_hbm, xr_hbm, src_hbm, dst_hbm, fused_hbm, we_hbm, att_hbm,
             out_hbm,
             src_v, dst_v, fz_v, xl_v, xr_v, row_v, we_v, att_v, io_v, acc_sh,
             sem1, sem2):
  cid = lax.axis_index("c")
  sid = lax.axis_index("s")
  wid = cid * NS + sid

  zvec = jnp.zeros((16,), jnp.float32)

  def zrow(j, zc):
    io_v[j, :] = zvec
    return zc

  lax.fori_loop(0, NR, zrow, 0, unroll=False)
  pltpu.sync_copy(io_v, acc_sh.at[pl.ds(sid * NR, NR)])
  pltpu.sync_copy(we_hbm, we_v)
  pltpu.sync_copy(att_hbm, att_v)
  plsc.subcore_barrier()

  iota = lax.iota(jnp.int32, 16)
  one = jnp.float32(1.0)
  zero = jnp.float32(0.0)
  dh = [jnp.where((iota == h) | (iota == h + 3), one, zero) for h in range(H)]
  oh = [jnp.where(iota == h + 3, one, zero) for h in range(H)]
  mask012 = jnp.where(iota < 3, one, zero)

  base0 = wid * EPW

  def chunk_body(ci, carry):
    base = base0 + ci * C
    pltpu.sync_copy(src_hbm.at[pl.ds(base, C)], src_v)
    pltpu.sync_copy(dst_hbm.at[pl.ds(base, C)], dst_v)
    pltpu.sync_copy(fused_hbm.at[pl.ds(base, C)], fz_v)
    cp1 = pltpu.async_copy(xl_hbm.at[src_v], xl_v, sem1)
    cp2 = pltpu.async_copy(xr_hbm.at[dst_v], xr_v, sem2)
    cp1.wait()
    cp2.wait()

    def edge_body(j, ecarry):
      fj = fz_v[j]
      heads = []
      for h in range(H):
        acc = None
        for k in range(HID // 16):
          off = h * HID + k * 16
          z = (xl_v[j, pl.ds(off, 16)] + xr_v[j, pl.ds(off, 16)]
               + fj * we_v[h, pl.ds(k * 16, 16)])
          zlr = jnp.where(z >= zero, z, z * jnp.float32(0.2))
          t = zlr * att_v[h, pl.ds(k * 16, 16)]
          acc = t if acc is None else acc + t
        heads.append(jnp.sum(acc))
      alpha_vec = heads[0] * dh[0] + heads[1] * dh[1] + heads[2] * dh[2]
      ex = jnp.exp(alpha_vec)
      slv16 = xl_v[j, pl.ds(H * HID, 16)]
      slv = slv16[0] * oh[0] + slv16[1] * oh[1] + slv16[2] * oh[2]
      row_v[j, :] = ex * (mask012 + slv)
      return ecarry

    lax.fori_loop(0, C, edge_body, 0, unroll=False)
    pltpu.sync_copy(row_v, acc_sh.at[dst_v], add=True)
    return carry

  lax.fori_loop(0, NCHUNK, chunk_body, 0, unroll=False)
  plsc.subcore_barrier()
  pltpu.sync_copy(acc_sh.at[pl.ds(sid * NR, NR)], io_v)
  pltpu.sync_copy(io_v, out_hbm.at[pl.ds(cid * NPAD + sid * NR, NR)])


def _final_body(acc_ref, bias_ref, linw_ref, linb_ref, out_ref):
  d = acc_ref[0:N] + acc_ref[NPAD:NPAD + N]
  den = d[:, 0:H]
  num = d[:, H:2 * H]
  ratio = num / (den + jnp.float32(1e-16))
  const = jnp.sum(bias_ref[...] * linw_ref[...]) + linb_ref[0, 0]
  scores = jnp.sum(ratio, axis=1, keepdims=True) / jnp.float32(H) + const
  m = jnp.max(scores)
  e = jnp.exp(scores - m)
  out_ref[...] = e / jnp.sum(e)


def kernel(edge_index, edge_attr, path_emb, sim_w, W_l, b_l, W_r, b_r, W_e,
           att, bias, lin_W, lin_b):
  src = edge_index[0]
  dst = edge_index[1]

  nblocks = N // NB
  xl_ext, xr_ext = pl.pallas_call(
      _prep_body,
      grid=(nblocks,),
      in_specs=[
          pl.BlockSpec((NB, EMB), lambda i: (i, 0)),
          pl.BlockSpec((EMB, H * HID), lambda i: (0, 0)),
          pl.BlockSpec((1, H * HID), lambda i: (0, 0)),
          pl.BlockSpec((EMB, H * HID), lambda i: (0, 0)),
          pl.BlockSpec((1, H * HID), lambda i: (0, 0)),
          pl.BlockSpec((EMB, 1), lambda i: (0, 0)),
      ],
      out_specs=[
          pl.BlockSpec((NB, ROWL), lambda i: (i, 0)),
          pl.BlockSpec((NB, ROWR), lambda i: (i, 0)),
      ],
      out_shape=[
          jax.ShapeDtypeStruct((N, ROWL), jnp.float32),
          jax.ShapeDtypeStruct((N, ROWR), jnp.float32),
      ],
  )(path_emb, W_l, b_l.reshape(1, -1), W_r, b_r.reshape(1, -1), lin_W)

  EB = 8000
  fused = pl.pallas_call(
      _fused_body,
      grid=(E // EB,),
      in_specs=[
          pl.BlockSpec((EB, edge_attr.shape[1]), lambda i: (i, 0)),
          pl.BlockSpec((1, sim_w.shape[0]), lambda i: (0, 0)),
      ],
      out_specs=pl.BlockSpec((EB, ACCW), lambda i: (i, 0)),
      out_shape=jax.ShapeDtypeStruct((E, ACCW), jnp.float32),
  )(edge_attr, sim_w.reshape(1, -1))

  we_r = W_e.reshape(H, HID)

  mesh = plsc.VectorSubcoreMesh(core_axis_name="c", subcore_axis_name="s",
                                num_cores=NC, num_subcores=NS)
  acc = pl.kernel(
      _sc_body,
      out_type=jax.ShapeDtypeStruct((NC * NPAD, ACCW), jnp.float32),
      mesh=mesh,
      compiler_params=pltpu.CompilerParams(needs_layout_passes=False,
                                           use_tc_tiling_on_sc=False),
      scratch_types=[
          pltpu.VMEM((C,), jnp.int32),
          pltpu.VMEM((C,), jnp.int32),
          pltpu.VMEM((C, ACCW), jnp.float32),
          pltpu.VMEM((C, ROWL), jnp.float32),
          pltpu.VMEM((C, ROWR), jnp.float32),
          pltpu.VMEM((C, ACCW), jnp.float32),
          pltpu.VMEM((H, HID), jnp.float32),
          pltpu.VMEM((H, HID), jnp.float32),
          pltpu.VMEM((NR, ACCW), jnp.float32),
          pltpu.VMEM_SHARED((NPAD, ACCW), jnp.float32),
          pltpu.SemaphoreType.DMA,
          pltpu.SemaphoreType.DMA,
      ],
  )(xl_ext, xr_ext, src, dst, fused, we_r, att)

  weights = pl.pallas_call(
      _final_body,
      in_specs=[
          pl.BlockSpec((NC * NPAD, ACCW), lambda: (0, 0)),
          pl.BlockSpec((1, HID), lambda: (0, 0)),
          pl.BlockSpec((1, HID), lambda: (0, 0)),
          pl.BlockSpec((1, 1), lambda: (0, 0)),
      ],
      out_specs=pl.BlockSpec((N, 1), lambda: (0, 0)),
      out_shape=jax.ShapeDtypeStruct((N, 1), jnp.float32),
  )(acc, bias.reshape(1, -1), lin_W.reshape(1, -1), lin_b.reshape(1, 1))
  return weights.reshape(N)
